# baseline (speedup 1.0000x reference)
import functools

import jax
import jax.numpy as jnp
from jax import lax
from jax.experimental import pallas as pl
from jax.experimental.pallas import tpu as pltpu
from jax.experimental.pallas import tpu_sc as plsc

_F = 26
_B = 4096
_CHUNK = 128
_TC = 512


@functools.cache
def _build_transpose(V, D):
    def tbody(xt_ref, out_ref):
        out_ref[...] = xt_ref[...].T

    return pl.pallas_call(
        tbody,
        grid=(pl.cdiv(V, _TC),),
        in_specs=[pl.BlockSpec((D, _TC), lambda i: (0, i))],
        out_specs=pl.BlockSpec((_TC, D), lambda i: (i, 0)),
        out_shape=jax.ShapeDtypeStruct((V, D), jnp.float32),
    )


@functools.cache
def _build(V, D, N):
    info = plsc.get_sparse_core_info()
    NC, NS, L = info.num_cores, info.num_subcores, info.num_lanes
    NW = NC * NS
    assert N % (NW * _CHUNK) == 0
    rpw = N // NW
    n_chunks = rpw // _CHUNK
    mesh = plsc.VectorSubcoreMesh(core_axis_name="c", subcore_axis_name="s")

    @functools.partial(
        pl.kernel,
        out_type=(
            jax.ShapeDtypeStruct((N, D), jnp.float32),
            jax.ShapeDtypeStruct((_F * L,), jnp.int32),
        ),
        mesh=mesh,
        compiler_params=pltpu.CompilerParams(use_tc_tiling_on_sc=False),
        scratch_types=[
            pltpu.VMEM((rpw,), jnp.int32),
            pltpu.VMEM((rpw, D), jnp.float32),
            pltpu.VMEM((_B,), jnp.int32),
            pltpu.VMEM((L,), jnp.int32),
            pltpu.SemaphoreType.DMA,
        ],
    )
    def gather_kernel(table_hbm, values_hbm, lengths_hbm, out_hbm, sums_hbm,
                      idx_v, rows_v, len_v, acc_v, gsem):
        wid = lax.axis_index("s") * NC + lax.axis_index("c")
        base = wid * rpw
        pltpu.sync_copy(values_hbm.at[pl.ds(base, rpw)], idx_v)
        for j in range(n_chunks):
            pltpu.async_copy(table_hbm.at[idx_v.at[pl.ds(j * _CHUNK, _CHUNK)]],
                             rows_v.at[pl.ds(j * _CHUNK, _CHUNK)], gsem)

        @pl.when(wid < _F)
        def _():
            pltpu.sync_copy(lengths_hbm.at[pl.ds(wid * _B, _B)], len_v)

            def step(i, acc):
                return acc + len_v[pl.ds(i * L, L)]

            acc_v[...] = lax.fori_loop(0, _B // L, step,
                                       jnp.zeros((L,), jnp.int32))
            pltpu.sync_copy(acc_v, sums_hbm.at[pl.ds(wid * L, L)])

        pltpu.make_async_copy(out_hbm.at[pl.ds(base, rpw)], rows_v,
                              gsem).wait()
        pltpu.sync_copy(rows_v, out_hbm.at[pl.ds(base, rpw)])

    return gather_kernel


@jax.jit
def kernel(table, values, lengths):
    V, D = table.shape
    N = values.shape[0]
    L = 16
    table_lin = _build_transpose(V, D)(table.T)
    rows, sums = _build(V, D, N)(table_lin, values, lengths)
    split_embeddings = rows.reshape(_F, _B, D)
    reduce_lengths = sums.reshape(_F, L).sum(axis=1)
    offsets = jnp.concatenate([
        jnp.zeros((1,), reduce_lengths.dtype),
        jnp.cumsum(reduce_lengths),
    ])
    split_lengths = lengths.reshape(_F, _B)
    return split_embeddings, split_lengths, offsets


# speedup vs baseline: 2.4397x; 2.4397x over previous
import functools

import jax
import jax.numpy as jnp
from jax import lax
from jax.experimental import pallas as pl
from jax.experimental.pallas import tpu as pltpu
from jax.experimental.pallas import tpu_sc as plsc

_F = 26
_B = 4096
_CHUNK = 128
_TC = 256
_Q = 250112


@functools.cache
def _build_transpose(V, D):
    n_blk = _Q // _TC

    def tbody(x0, x1, x2, x3, out_ref):
        x = jnp.concatenate(
            [x0[...], x1[...], x2[...], x3[...]], axis=0)
        out_ref[...] = x.T

    last_blk = (V + _TC - 1) // _TC - 1

    def in_spec(a):
        return pl.BlockSpec(
            (D, _TC),
            lambda i, a=a: (0, jnp.minimum(a * n_blk + i, last_blk)))

    return pl.pallas_call(
        tbody,
        grid=(n_blk,),
        in_specs=[in_spec(a) for a in range(4)],
        out_specs=pl.BlockSpec((_TC, 4 * D), lambda i: (i, 0)),
        out_shape=jax.ShapeDtypeStruct((_Q, 4 * D), jnp.float32),
    )


@functools.cache
def _build(V, D, N):
    info = plsc.get_sparse_core_info()
    NC, NS, L = info.num_cores, info.num_subcores, info.num_lanes
    NW = NC * NS
    assert N % (NW * _CHUNK) == 0
    rpw = N // NW
    n_chunks = rpw // _CHUNK
    mesh = plsc.VectorSubcoreMesh(core_axis_name="c", subcore_axis_name="s")

    @functools.partial(
        pl.kernel,
        out_type=(
            jax.ShapeDtypeStruct((N, D), jnp.float32),
            jax.ShapeDtypeStruct((_F * L,), jnp.int32),
        ),
        mesh=mesh,
        compiler_params=pltpu.CompilerParams(use_tc_tiling_on_sc=False),
        scratch_types=[
            pltpu.VMEM((rpw,), jnp.int32),
            pltpu.VMEM((rpw, D), jnp.float32),
            pltpu.VMEM((_B,), jnp.int32),
            pltpu.VMEM((L,), jnp.int32),
            pltpu.SemaphoreType.DMA,
        ],
    )
    def gather_kernel(table_hbm, values_hbm, lengths_hbm, out_hbm, sums_hbm,
                      idx_v, rows_v, len_v, acc_v, gsem):
        wid = lax.axis_index("s") * NC + lax.axis_index("c")
        base = wid * rpw
        pltpu.sync_copy(values_hbm.at[pl.ds(base, rpw)], idx_v)

        def xform(s, _):
            x = idx_v[pl.ds(s * L, L)]
            neg = (lax.shift_right_logical(x - _Q, 31)
                   + lax.shift_right_logical(x - 2 * _Q, 31)
                   + lax.shift_right_logical(x - 3 * _Q, 31))
            a = 3 - neg
            idx_v[pl.ds(s * L, L)] = x * 4 + a * (1 - 4 * _Q)
            return 0

        lax.fori_loop(0, rpw // L, xform, 0)
        for j in range(n_chunks):
            pltpu.async_copy(table_hbm.at[idx_v.at[pl.ds(j * _CHUNK, _CHUNK)]],
                             rows_v.at[pl.ds(j * _CHUNK, _CHUNK)], gsem)

        @pl.when(wid < _F)
        def _():
            pltpu.sync_copy(lengths_hbm.at[pl.ds(wid * _B, _B)], len_v)

            def step(i, acc):
                return acc + len_v[pl.ds(i * L, L)]

            acc_v[...] = lax.fori_loop(0, _B // L, step,
                                       jnp.zeros((L,), jnp.int32))
            pltpu.sync_copy(acc_v, sums_hbm.at[pl.ds(wid * L, L)])

        pltpu.make_async_copy(out_hbm.at[pl.ds(base, rpw)], rows_v,
                              gsem).wait()
        pltpu.sync_copy(rows_v, out_hbm.at[pl.ds(base, rpw)])

    return gather_kernel


@jax.jit
def kernel(table, values, lengths):
    V, D = table.shape
    N = values.shape[0]
    L = 16
    tt = table.T
    packed = _build_transpose(V, D)(tt, tt, tt, tt)
    pseudo = packed.reshape(4 * _Q, D)
    rows, sums = _build(4 * _Q, D, N)(pseudo, values, lengths)
    split_embeddings = rows.reshape(_F, _B, D)
    reduce_lengths = sums.reshape(_F, L).sum(axis=1)
    offsets = jnp.concatenate([
        jnp.zeros((1,), reduce_lengths.dtype),
        jnp.cumsum(reduce_lengths),
    ])
    split_lengths = lengths.reshape(_F, _B)
    return split_embeddings, split_lengths, offsets


# speedup vs baseline: 5.7377x; 2.3518x over previous
import functools

import jax
import jax.numpy as jnp
from jax import lax
from jax.experimental import pallas as pl
from jax.experimental.pallas import tpu as pltpu
from jax.experimental.pallas import tpu_sc as plsc

_F = 26
_B = 4096
_CHUNK = 128
_TC = 1024
_Q = 250880


@functools.cache
def _build_transpose(V, D):
    n_blk = _Q // _TC

    def tbody(x0, x1, x2, x3, out_ref):
        x = jnp.concatenate(
            [x0[...], x1[...], x2[...], x3[...]], axis=0)
        out_ref[...] = x.T

    last_blk = (V + _TC - 1) // _TC - 1

    def in_spec(a):
        return pl.BlockSpec(
            (D, _TC),
            lambda i, a=a: (0, jnp.minimum(a * n_blk + i, last_blk)))

    return pl.pallas_call(
        tbody,
        grid=(n_blk,),
        in_specs=[in_spec(a) for a in range(4)],
        out_specs=pl.BlockSpec((_TC, 4 * D), lambda i: (i, 0)),
        out_shape=jax.ShapeDtypeStruct((_Q, 4 * D), jnp.float32),
    )


@functools.cache
def _build(V, D, N):
    info = plsc.get_sparse_core_info()
    NC, NS, L = info.num_cores, info.num_subcores, info.num_lanes
    NW = NC * NS
    assert N % (NW * _CHUNK) == 0
    rpw = N // NW
    n_chunks = rpw // _CHUNK
    mesh = plsc.VectorSubcoreMesh(core_axis_name="c", subcore_axis_name="s")

    @functools.partial(
        pl.kernel,
        out_type=(
            jax.ShapeDtypeStruct((N, D), jnp.float32),
            jax.ShapeDtypeStruct((_F * L,), jnp.int32),
        ),
        mesh=mesh,
        compiler_params=pltpu.CompilerParams(use_tc_tiling_on_sc=False),
        scratch_types=[
            pltpu.VMEM((rpw,), jnp.int32),
            pltpu.VMEM((rpw, D), jnp.float32),
            pltpu.VMEM((_B,), jnp.int32),
            pltpu.VMEM((L,), jnp.int32),
            pltpu.SemaphoreType.DMA,
        ],
    )
    def gather_kernel(table_hbm, values_hbm, lengths_hbm, out_hbm, sums_hbm,
                      idx_v, rows_v, len_v, acc_v, gsem):
        wid = lax.axis_index("s") * NC + lax.axis_index("c")
        base = wid * rpw
        pltpu.sync_copy(values_hbm.at[pl.ds(base, rpw)], idx_v)

        def xform(s, _):
            x = idx_v[pl.ds(s * L, L)]
            neg = (lax.shift_right_logical(x - _Q, 31)
                   + lax.shift_right_logical(x - 2 * _Q, 31)
                   + lax.shift_right_logical(x - 3 * _Q, 31))
            a = 3 - neg
            idx_v[pl.ds(s * L, L)] = x * 4 + a * (1 - 4 * _Q)
            return 0

        lax.fori_loop(0, rpw // L, xform, 0)
        for j in range(n_chunks):
            pltpu.async_copy(table_hbm.at[idx_v.at[pl.ds(j * _CHUNK, _CHUNK)]],
                             rows_v.at[pl.ds(j * _CHUNK, _CHUNK)], gsem)

        @pl.when(wid < _F)
        def _():
            pltpu.sync_copy(lengths_hbm.at[pl.ds(wid * _B, _B)], len_v)

            def step(i, acc):
                return acc + len_v[pl.ds(i * L, L)]

            acc_v[...] = lax.fori_loop(0, _B // L, step,
                                       jnp.zeros((L,), jnp.int32))
            pltpu.sync_copy(acc_v, sums_hbm.at[pl.ds(wid * L, L)])

        pltpu.make_async_copy(out_hbm.at[pl.ds(base, rpw)], rows_v,
                              gsem).wait()
        pltpu.sync_copy(rows_v, out_hbm.at[pl.ds(base, rpw)])

    return gather_kernel


@jax.jit
def kernel(table, values, lengths):
    V, D = table.shape
    N = values.shape[0]
    L = 16
    tt = table.T
    packed = _build_transpose(V, D)(tt, tt, tt, tt)
    pseudo = packed.reshape(4 * _Q, D)
    rows, sums = _build(4 * _Q, D, N)(pseudo, values, lengths)
    split_embeddings = rows.reshape(_F, _B, D)
    reduce_lengths = sums.reshape(_F, L).sum(axis=1)
    offsets = jnp.concatenate([
        jnp.zeros((1,), reduce_lengths.dtype),
        jnp.cumsum(reduce_lengths),
    ])
    split_lengths = lengths.reshape(_F, _B)
    return split_embeddings, split_lengths, offsets


# speedup vs baseline: 7.2838x; 1.2695x over previous
import functools

import jax
import jax.numpy as jnp
from jax import lax
from jax.experimental import pallas as pl
from jax.experimental.pallas import tpu as pltpu
from jax.experimental.pallas import tpu_sc as plsc

_F = 26
_B = 4096
_CHUNK = 128
_TC = 2048
_Q = 251904


@functools.cache
def _build_transpose(V, D):
    n_blk = _Q // _TC

    def tbody(x0, x1, x2, x3, out_ref):
        x = jnp.concatenate(
            [x0[...], x1[...], x2[...], x3[...]], axis=0)
        out_ref[...] = x.T

    last_blk = (V + _TC - 1) // _TC - 1

    def in_spec(a):
        return pl.BlockSpec(
            (D, _TC),
            lambda i, a=a: (0, jnp.minimum(a * n_blk + i, last_blk)))

    return pl.pallas_call(
        tbody,
        grid=(n_blk,),
        in_specs=[in_spec(a) for a in range(4)],
        out_specs=pl.BlockSpec((_TC, 4 * D), lambda i: (i, 0)),
        out_shape=jax.ShapeDtypeStruct((_Q, 4 * D), jnp.float32),
    )


@functools.cache
def _build(V, D, N):
    info = plsc.get_sparse_core_info()
    NC, NS, L = info.num_cores, info.num_subcores, info.num_lanes
    NW = NC * NS
    assert N % (NW * _CHUNK) == 0
    rpw = N // NW
    n_chunks = rpw // _CHUNK
    mesh = plsc.VectorSubcoreMesh(core_axis_name="c", subcore_axis_name="s")

    @functools.partial(
        pl.kernel,
        out_type=(
            jax.ShapeDtypeStruct((N, D), jnp.float32),
            jax.ShapeDtypeStruct((_F * L,), jnp.int32),
        ),
        mesh=mesh,
        compiler_params=pltpu.CompilerParams(use_tc_tiling_on_sc=False),
        scratch_types=[
            pltpu.VMEM((rpw,), jnp.int32),
            pltpu.VMEM((rpw, D), jnp.float32),
            pltpu.VMEM((_B,), jnp.int32),
            pltpu.VMEM((L,), jnp.int32),
            pltpu.SemaphoreType.DMA,
        ],
    )
    def gather_kernel(table_hbm, values_hbm, lengths_hbm, out_hbm, sums_hbm,
                      idx_v, rows_v, len_v, acc_v, gsem):
        wid = lax.axis_index("s") * NC + lax.axis_index("c")
        base = wid * rpw
        pltpu.sync_copy(values_hbm.at[pl.ds(base, rpw)], idx_v)

        def xform(s, _):
            x = idx_v[pl.ds(s * L, L)]
            neg = (lax.shift_right_logical(x - _Q, 31)
                   + lax.shift_right_logical(x - 2 * _Q, 31)
                   + lax.shift_right_logical(x - 3 * _Q, 31))
            a = 3 - neg
            idx_v[pl.ds(s * L, L)] = x * 4 + a * (1 - 4 * _Q)
            return 0

        lax.fori_loop(0, rpw // L, xform, 0)
        for j in range(n_chunks):
            pltpu.async_copy(table_hbm.at[idx_v.at[pl.ds(j * _CHUNK, _CHUNK)]],
                             rows_v.at[pl.ds(j * _CHUNK, _CHUNK)], gsem)

        @pl.when(wid < _F)
        def _():
            pltpu.sync_copy(lengths_hbm.at[pl.ds(wid * _B, _B)], len_v)

            def step(i, acc):
                return acc + len_v[pl.ds(i * L, L)]

            acc_v[...] = lax.fori_loop(0, _B // L, step,
                                       jnp.zeros((L,), jnp.int32))
            pltpu.sync_copy(acc_v, sums_hbm.at[pl.ds(wid * L, L)])

        pltpu.make_async_copy(out_hbm.at[pl.ds(base, rpw)], rows_v,
                              gsem).wait()
        pltpu.sync_copy(rows_v, out_hbm.at[pl.ds(base, rpw)])

    return gather_kernel


@jax.jit
def kernel(table, values, lengths):
    V, D = table.shape
    N = values.shape[0]
    L = 16
    tt = table.T
    packed = _build_transpose(V, D)(tt, tt, tt, tt)
    pseudo = packed.reshape(4 * _Q, D)
    rows, sums = _build(4 * _Q, D, N)(pseudo, values, lengths)
    split_embeddings = rows.reshape(_F, _B, D)
    reduce_lengths = sums.reshape(_F, L).sum(axis=1)
    offsets = jnp.concatenate([
        jnp.zeros((1,), reduce_lengths.dtype),
        jnp.cumsum(reduce_lengths),
    ])
    split_lengths = lengths.reshape(_F, _B)
    return split_embeddings, split_lengths, offsets


# speedup vs baseline: 8.7434x; 1.2004x over previous
import functools

import jax
import jax.numpy as jnp
from jax import lax
from jax.experimental import pallas as pl
from jax.experimental.pallas import tpu as pltpu
from jax.experimental.pallas import tpu_sc as plsc

_F = 26
_B = 4096
_CHUNK = 128
_TC = 4096
_Q = 253952


@functools.cache
def _build_transpose(V, D):
    n_blk = _Q // _TC

    def tbody(x0, x1, x2, x3, out_ref):
        x = jnp.concatenate(
            [x0[...], x1[...], x2[...], x3[...]], axis=0)
        out_ref[...] = x.T

    last_blk = (V + _TC - 1) // _TC - 1

    def in_spec(a):
        return pl.BlockSpec(
            (D, _TC),
            lambda i, a=a: (0, jnp.minimum(a * n_blk + i, last_blk)))

    return pl.pallas_call(
        tbody,
        grid=(n_blk,),
        in_specs=[in_spec(a) for a in range(4)],
        out_specs=pl.BlockSpec((_TC, 4 * D), lambda i: (i, 0)),
        out_shape=jax.ShapeDtypeStruct((_Q, 4 * D), jnp.float32),
    )


@functools.cache
def _build(V, D, N):
    info = plsc.get_sparse_core_info()
    NC, NS, L = info.num_cores, info.num_subcores, info.num_lanes
    NW = NC * NS
    assert N % (NW * _CHUNK) == 0
    rpw = N // NW
    n_chunks = rpw // _CHUNK
    mesh = plsc.VectorSubcoreMesh(core_axis_name="c", subcore_axis_name="s")

    @functools.partial(
        pl.kernel,
        out_type=(
            jax.ShapeDtypeStruct((N, D), jnp.float32),
            jax.ShapeDtypeStruct((_F * L,), jnp.int32),
        ),
        mesh=mesh,
        compiler_params=pltpu.CompilerParams(use_tc_tiling_on_sc=False),
        scratch_types=[
            pltpu.VMEM((rpw,), jnp.int32),
            pltpu.VMEM((rpw, D), jnp.float32),
            pltpu.VMEM((_B,), jnp.int32),
            pltpu.VMEM((L,), jnp.int32),
            pltpu.SemaphoreType.DMA,
        ],
    )
    def gather_kernel(table_hbm, values_hbm, lengths_hbm, out_hbm, sums_hbm,
                      idx_v, rows_v, len_v, acc_v, gsem):
        wid = lax.axis_index("s") * NC + lax.axis_index("c")
        base = wid * rpw
        pltpu.sync_copy(values_hbm.at[pl.ds(base, rpw)], idx_v)

        def xform(s, _):
            x = idx_v[pl.ds(s * L, L)]
            neg = (lax.shift_right_logical(x - _Q, 31)
                   + lax.shift_right_logical(x - 2 * _Q, 31)
                   + lax.shift_right_logical(x - 3 * _Q, 31))
            a = 3 - neg
            idx_v[pl.ds(s * L, L)] = x * 4 + a * (1 - 4 * _Q)
            return 0

        lax.fori_loop(0, rpw // L, xform, 0)
        for j in range(n_chunks):
            pltpu.async_copy(table_hbm.at[idx_v.at[pl.ds(j * _CHUNK, _CHUNK)]],
                             rows_v.at[pl.ds(j * _CHUNK, _CHUNK)], gsem)

        @pl.when(wid < _F)
        def _():
            pltpu.sync_copy(lengths_hbm.at[pl.ds(wid * _B, _B)], len_v)

            def step(i, acc):
                return acc + len_v[pl.ds(i * L, L)]

            acc_v[...] = lax.fori_loop(0, _B // L, step,
                                       jnp.zeros((L,), jnp.int32))
            pltpu.sync_copy(acc_v, sums_hbm.at[pl.ds(wid * L, L)])

        pltpu.make_async_copy(out_hbm.at[pl.ds(base, rpw)], rows_v,
                              gsem).wait()
        pltpu.sync_copy(rows_v, out_hbm.at[pl.ds(base, rpw)])

    return gather_kernel


@jax.jit
def kernel(table, values, lengths):
    V, D = table.shape
    N = values.shape[0]
    L = 16
    tt = table.T
    packed = _build_transpose(V, D)(tt, tt, tt, tt)
    pseudo = packed.reshape(4 * _Q, D)
    rows, sums = _build(4 * _Q, D, N)(pseudo, values, lengths)
    split_embeddings = rows.reshape(_F, _B, D)
    reduce_lengths = sums.reshape(_F, L).sum(axis=1)
    offsets = jnp.concatenate([
        jnp.zeros((1,), reduce_lengths.dtype),
        jnp.cumsum(reduce_lengths),
    ])
    split_lengths = lengths.reshape(_F, _B)
    return split_embeddings, split_lengths, offsets


# speedup vs baseline: 9.4971x; 1.0862x over previous
import functools

import jax
import jax.numpy as jnp
from jax import lax
from jax.experimental import pallas as pl
from jax.experimental.pallas import tpu as pltpu
from jax.experimental.pallas import tpu_sc as plsc

_F = 26
_B = 4096
_CHUNK = 128
_TC = 8192
_Q = 253952


@functools.cache
def _build_transpose(V, D):
    n_blk = _Q // _TC

    def tbody(x0, x1, x2, x3, out_ref):
        x = jnp.concatenate(
            [x0[...], x1[...], x2[...], x3[...]], axis=0)
        out_ref[...] = x.T

    last_blk = (V + _TC - 1) // _TC - 1

    def in_spec(a):
        return pl.BlockSpec(
            (D, _TC),
            lambda i, a=a: (0, jnp.minimum(a * n_blk + i, last_blk)))

    return pl.pallas_call(
        tbody,
        grid=(n_blk,),
        in_specs=[in_spec(a) for a in range(4)],
        out_specs=pl.BlockSpec((_TC, 4 * D), lambda i: (i, 0)),
        out_shape=jax.ShapeDtypeStruct((_Q, 4 * D), jnp.float32),
    )


@functools.cache
def _build(V, D, N):
    info = plsc.get_sparse_core_info()
    NC, NS, L = info.num_cores, info.num_subcores, info.num_lanes
    NW = NC * NS
    assert N % (NW * _CHUNK) == 0
    rpw = N // NW
    n_chunks = rpw // _CHUNK
    mesh = plsc.VectorSubcoreMesh(core_axis_name="c", subcore_axis_name="s")

    @functools.partial(
        pl.kernel,
        out_type=(
            jax.ShapeDtypeStruct((N, D), jnp.float32),
            jax.ShapeDtypeStruct((_F * L,), jnp.int32),
        ),
        mesh=mesh,
        compiler_params=pltpu.CompilerParams(use_tc_tiling_on_sc=False),
        scratch_types=[
            pltpu.VMEM((rpw,), jnp.int32),
            pltpu.VMEM((rpw, D), jnp.float32),
            pltpu.VMEM((_B,), jnp.int32),
            pltpu.VMEM((L,), jnp.int32),
            pltpu.SemaphoreType.DMA,
        ],
    )
    def gather_kernel(table_hbm, values_hbm, lengths_hbm, out_hbm, sums_hbm,
                      idx_v, rows_v, len_v, acc_v, gsem):
        wid = lax.axis_index("s") * NC + lax.axis_index("c")
        base = wid * rpw
        pltpu.sync_copy(values_hbm.at[pl.ds(base, rpw)], idx_v)

        def xform(s, _):
            x = idx_v[pl.ds(s * L, L)]
            neg = (lax.shift_right_logical(x - _Q, 31)
                   + lax.shift_right_logical(x - 2 * _Q, 31)
                   + lax.shift_right_logical(x - 3 * _Q, 31))
            a = 3 - neg
            idx_v[pl.ds(s * L, L)] = x * 4 + a * (1 - 4 * _Q)
            return 0

        lax.fori_loop(0, rpw // L, xform, 0)
        for j in range(n_chunks):
            pltpu.async_copy(table_hbm.at[idx_v.at[pl.ds(j * _CHUNK, _CHUNK)]],
                             rows_v.at[pl.ds(j * _CHUNK, _CHUNK)], gsem)

        @pl.when(wid < _F)
        def _():
            pltpu.sync_copy(lengths_hbm.at[pl.ds(wid * _B, _B)], len_v)

            def step(i, acc):
                return acc + len_v[pl.ds(i * L, L)]

            acc_v[...] = lax.fori_loop(0, _B // L, step,
                                       jnp.zeros((L,), jnp.int32))
            pltpu.sync_copy(acc_v, sums_hbm.at[pl.ds(wid * L, L)])

        pltpu.make_async_copy(out_hbm.at[pl.ds(base, rpw)], rows_v,
                              gsem).wait()
        pltpu.sync_copy(rows_v, out_hbm.at[pl.ds(base, rpw)])

    return gather_kernel


@jax.jit
def kernel(table, values, lengths):
    V, D = table.shape
    N = values.shape[0]
    L = 16
    tt = table.T
    packed = _build_transpose(V, D)(tt, tt, tt, tt)
    pseudo = packed.reshape(4 * _Q, D)
    rows, sums = _build(4 * _Q, D, N)(pseudo, values, lengths)
    split_embeddings = rows.reshape(_F, _B, D)
    reduce_lengths = sums.reshape(_F, L).sum(axis=1)
    offsets = jnp.concatenate([
        jnp.zeros((1,), reduce_lengths.dtype),
        jnp.cumsum(reduce_lengths),
    ])
    split_lengths = lengths.reshape(_F, _B)
    return split_embeddings, split_lengths, offsets


# speedup vs baseline: 9.5378x; 1.0043x over previous
import functools

import jax
import jax.numpy as jnp
from jax import lax
from jax.experimental import pallas as pl
from jax.experimental.pallas import tpu as pltpu
from jax.experimental.pallas import tpu_sc as plsc

_F = 26
_B = 4096
_CHUNK = 128
_TC = 16384
_Q = 262144


@functools.cache
def _build_transpose(V, D):
    n_blk = _Q // _TC

    def tbody(x0, x1, x2, x3, out_ref):
        x = jnp.concatenate(
            [x0[...], x1[...], x2[...], x3[...]], axis=0)
        out_ref[...] = x.T

    last_blk = (V + _TC - 1) // _TC - 1

    def in_spec(a):
        return pl.BlockSpec(
            (D, _TC),
            lambda i, a=a: (0, jnp.minimum(a * n_blk + i, last_blk)))

    return pl.pallas_call(
        tbody,
        grid=(n_blk,),
        in_specs=[in_spec(a) for a in range(4)],
        out_specs=pl.BlockSpec((_TC, 4 * D), lambda i: (i, 0)),
        out_shape=jax.ShapeDtypeStruct((_Q, 4 * D), jnp.float32),
    )


@functools.cache
def _build(V, D, N):
    info = plsc.get_sparse_core_info()
    NC, NS, L = info.num_cores, info.num_subcores, info.num_lanes
    NW = NC * NS
    assert N % (NW * _CHUNK) == 0
    rpw = N // NW
    n_chunks = rpw // _CHUNK
    mesh = plsc.VectorSubcoreMesh(core_axis_name="c", subcore_axis_name="s")

    @functools.partial(
        pl.kernel,
        out_type=(
            jax.ShapeDtypeStruct((N, D), jnp.float32),
            jax.ShapeDtypeStruct((_F * L,), jnp.int32),
        ),
        mesh=mesh,
        compiler_params=pltpu.CompilerParams(use_tc_tiling_on_sc=False),
        scratch_types=[
            pltpu.VMEM((rpw,), jnp.int32),
            pltpu.VMEM((rpw, D), jnp.float32),
            pltpu.VMEM((_B,), jnp.int32),
            pltpu.VMEM((L,), jnp.int32),
            pltpu.SemaphoreType.DMA,
        ],
    )
    def gather_kernel(table_hbm, values_hbm, lengths_hbm, out_hbm, sums_hbm,
                      idx_v, rows_v, len_v, acc_v, gsem):
        wid = lax.axis_index("s") * NC + lax.axis_index("c")
        base = wid * rpw
        pltpu.sync_copy(values_hbm.at[pl.ds(base, rpw)], idx_v)

        def xform(s, _):
            x = idx_v[pl.ds(s * L, L)]
            neg = (lax.shift_right_logical(x - _Q, 31)
                   + lax.shift_right_logical(x - 2 * _Q, 31)
                   + lax.shift_right_logical(x - 3 * _Q, 31))
            a = 3 - neg
            idx_v[pl.ds(s * L, L)] = x * 4 + a * (1 - 4 * _Q)
            return 0

        lax.fori_loop(0, rpw // L, xform, 0)
        for j in range(n_chunks):
            pltpu.async_copy(table_hbm.at[idx_v.at[pl.ds(j * _CHUNK, _CHUNK)]],
                             rows_v.at[pl.ds(j * _CHUNK, _CHUNK)], gsem)

        @pl.when(wid < _F)
        def _():
            pltpu.sync_copy(lengths_hbm.at[pl.ds(wid * _B, _B)], len_v)

            def step(i, acc):
                return acc + len_v[pl.ds(i * L, L)]

            acc_v[...] = lax.fori_loop(0, _B // L, step,
                                       jnp.zeros((L,), jnp.int32))
            pltpu.sync_copy(acc_v, sums_hbm.at[pl.ds(wid * L, L)])

        pltpu.make_async_copy(out_hbm.at[pl.ds(base, rpw)], rows_v,
                              gsem).wait()
        pltpu.sync_copy(rows_v, out_hbm.at[pl.ds(base, rpw)])

    return gather_kernel


@jax.jit
def kernel(table, values, lengths):
    V, D = table.shape
    N = values.shape[0]
    L = 16
    tt = table.T
    packed = _build_transpose(V, D)(tt, tt, tt, tt)
    pseudo = packed.reshape(4 * _Q, D)
    rows, sums = _build(4 * _Q, D, N)(pseudo, values, lengths)
    split_embeddings = rows.reshape(_F, _B, D)
    reduce_lengths = sums.reshape(_F, L).sum(axis=1)
    offsets = jnp.concatenate([
        jnp.zeros((1,), reduce_lengths.dtype),
        jnp.cumsum(reduce_lengths),
    ])
    split_lengths = lengths.reshape(_F, _B)
    return split_embeddings, split_lengths, offsets
